# baseline (speedup 1.0000x reference)
import functools

import jax
import jax.numpy as jnp
from jax import lax
from jax.experimental import pallas as pl
import jax.experimental.pallas.tpu as pltpu
from jax.experimental.pallas import tpu_sc as plsc

BB = 4096
_NC, _NS, _L = 2, 16, 16
_NW = _NC * _NS
_K = -0.72134752044448170368


def _sc_prep(mu_hbm, sg_hbm, idx_hbm, v_hbm, mu_v, sg_v, idx_v, out_v):
    wid = lax.axis_index("s") * _NC + lax.axis_index("c")
    r0 = wid * 2
    pltpu.sync_copy(mu_hbm, mu_v)
    pltpu.sync_copy(sg_hbm, sg_v)
    pltpu.sync_copy(idx_hbm.at[pl.ds(r0 * 64, 2 * 64)], idx_v)
    for rl in range(2):
        csum = jnp.zeros((_L,), jnp.float32)
        for j in range(64 // _L):
            col = lax.iota(jnp.int32, _L) + (j * _L)
            row = idx_v[pl.ds(rl * 64 + j * _L, _L)]
            flat = row * 64 + col
            mu_g = plsc.load_gather(mu_v, [flat])
            s_g = plsc.load_gather(sg_v, [flat])
            s_g = jnp.maximum(s_g, 1e-15)
            w = 1.0 / (s_g * s_g)
            a = mu_g * w
            out_v[pl.ds(rl * 192 + j * _L, _L)] = _K * w
            out_v[pl.ds(rl * 192 + 64 + j * _L, _L)] = (-2.0 * _K) * a
            csum = csum + mu_g * a
        cpart = (0.25 * _K) * csum
        for j in range(64 // _L):
            out_v[pl.ds(rl * 192 + 128 + j * _L, _L)] = cpart
    pltpu.sync_copy(out_v, v_hbm.at[pl.ds(r0 * 192, 2 * 192)])


def _tc_body(x_ref, v_ref, out_ref):
    x = x_ref[...]
    u = jnp.concatenate([x * x, x, jnp.ones_like(x)], axis=1)
    s = lax.dot_general(
        u, v_ref[...], (((1,), (1,)), ((), ())),
        preferred_element_type=jnp.float32,
        precision=lax.Precision.HIGHEST,
    )
    out_ref[...] = jnp.exp2(jnp.minimum(s, 0.0))


@jax.jit
def kernel(x, mu, sigma, rule_masks):
    b, i = x.shape
    r = rule_masks.shape[0]
    mesh = plsc.VectorSubcoreMesh(
        core_axis_name="c", subcore_axis_name="s",
        num_cores=_NC, num_subcores=_NS,
    )
    m = mu.shape[0]
    v = pl.kernel(
        _sc_prep,
        out_type=jax.ShapeDtypeStruct((r * 3 * i,), jnp.float32),
        mesh=mesh,
        compiler_params=pltpu.CompilerParams(needs_layout_passes=False),
        scratch_types=[
            pltpu.VMEM((m * i,), jnp.float32),
            pltpu.VMEM((m * i,), jnp.float32),
            pltpu.VMEM((2 * i,), jnp.int32),
            pltpu.VMEM((2 * 3 * i,), jnp.float32),
        ],
    )(
        mu.reshape(-1),
        sigma.reshape(-1),
        rule_masks.astype(jnp.int32).reshape(-1),
    ).reshape(r, 3 * i)
    return pl.pallas_call(
        _tc_body,
        grid=(b // BB,),
        in_specs=[
            pl.BlockSpec((BB, i), lambda j: (j, 0)),
            pl.BlockSpec((r, 3 * i), lambda j: (0, 0)),
        ],
        out_specs=pl.BlockSpec((BB, r), lambda j: (j, 0)),
        out_shape=jax.ShapeDtypeStruct((b, r), jnp.float32),
    )(x, v)


# speedup vs baseline: 2.1798x; 2.1798x over previous
import functools

import jax
import jax.numpy as jnp
from jax.experimental import pallas as pl
import jax.experimental.pallas.tpu as pltpu

BB = 4096


def _fuzzy_kernel(x_ref, mu_ref, sigma_ref, idx_ref, out_ref, w_ref, a_ref, c_ref):
    @pl.when(pl.program_id(0) == 0)
    def _prep():
        mu = mu_ref[...]
        sg = sigma_ref[...]
        idx = idx_ref[...]
        m = mu.shape[0]
        iota = jax.lax.broadcasted_iota(jnp.int32, (m,) + idx.shape, 0)
        onehot = (iota == idx[None, :, :]).astype(jnp.float32)
        mu_g = jnp.sum(onehot * mu[:, None, :], axis=0)
        sg_g = jnp.sum(onehot * sg[:, None, :], axis=0)
        sg_g = jnp.maximum(sg_g, 1e-15)
        w = 1.0 / (sg_g * sg_g)
        a = mu_g * w
        k = -0.72134752044448170368
        w_ref[...] = k * w
        a_ref[...] = (-2.0 * k) * a
        q = mu_g * a
        c_ref[...] = k * jax.lax.dot_general(
            jnp.ones((1, q.shape[1]), jnp.float32), q,
            (((1,), (1,)), ((), ())),
            preferred_element_type=jnp.float32,
            precision=jax.lax.Precision.HIGHEST,
        )

    x = x_ref[...]
    dims = (((1,), (1,)), ((), ()))
    s = (
        jax.lax.dot_general(
            x * x, w_ref[...], dims,
            preferred_element_type=jnp.float32,
            precision=jax.lax.Precision.HIGHEST,
        )
        + jax.lax.dot_general(
            x, a_ref[...], dims,
            preferred_element_type=jnp.float32,
            precision=jax.lax.Precision.HIGHEST,
        )
        + c_ref[...]
    )
    out_ref[...] = jnp.exp2(jnp.minimum(s, 0.0))


@functools.partial(jax.jit, static_argnames=("interpret",))
def kernel(x, mu, sigma, rule_masks, interpret=False):
    b, i = x.shape
    r = rule_masks.shape[0]
    grid = (b // BB,)
    return pl.pallas_call(
        _fuzzy_kernel,
        grid=grid,
        in_specs=[
            pl.BlockSpec((BB, i), lambda j: (j, 0)),
            pl.BlockSpec(mu.shape, lambda j: (0, 0)),
            pl.BlockSpec(sigma.shape, lambda j: (0, 0)),
            pl.BlockSpec(rule_masks.shape, lambda j: (0, 0)),
        ],
        out_specs=pl.BlockSpec((BB, r), lambda j: (j, 0)),
        out_shape=jax.ShapeDtypeStruct((b, r), jnp.float32),
        scratch_shapes=[
            pltpu.VMEM((r, i), jnp.float32),
            pltpu.VMEM((r, i), jnp.float32),
            pltpu.VMEM((1, r), jnp.float32),
        ],
        interpret=interpret,
    )(x, mu, sigma, rule_masks.astype(jnp.int32))


# speedup vs baseline: 2.7057x; 1.2413x over previous
import jax
import jax.numpy as jnp
from jax.experimental import pallas as pl
import jax.experimental.pallas.tpu as pltpu

BB = 4096


def _fuzzy_kernel(x_ref, mu_ref, sigma_ref, idx_ref, out_ref, v_ref, c_ref):
    @pl.when(pl.program_id(0) == 0)
    def _prep():
        mu = mu_ref[...]
        sg = sigma_ref[...]
        idx = idx_ref[...]
        m = mu.shape[0]
        iota = jax.lax.broadcasted_iota(jnp.int32, (m,) + idx.shape, 0)
        onehot = (iota == idx[None, :, :]).astype(jnp.float32)
        mu_g = jnp.sum(onehot * mu[:, None, :], axis=0)
        sg_g = jnp.sum(onehot * sg[:, None, :], axis=0)
        sg_g = jnp.maximum(sg_g, 1e-15)
        w = 1.0 / (sg_g * sg_g)
        a = mu_g * w
        k = -0.72134752044448170368
        v_ref[...] = jnp.concatenate([k * w, (-2.0 * k) * a], axis=1)
        q = mu_g * a
        c_ref[...] = k * jax.lax.dot_general(
            jnp.ones((1, q.shape[1]), jnp.float32), q,
            (((1,), (1,)), ((), ())),
            preferred_element_type=jnp.float32,
            precision=jax.lax.Precision.HIGHEST,
        )

    x = x_ref[...]
    u = jnp.concatenate([x * x, x], axis=1)
    s = jax.lax.dot_general(
        u, v_ref[...], (((1,), (1,)), ((), ())),
        preferred_element_type=jnp.float32,
        precision=jax.lax.Precision.HIGHEST,
    ) + c_ref[...]
    out_ref[...] = jnp.exp2(jnp.minimum(s, 0.0))


@jax.jit
def kernel(x, mu, sigma, rule_masks):
    b, i = x.shape
    r = rule_masks.shape[0]
    grid = (b // BB,)
    return pl.pallas_call(
        _fuzzy_kernel,
        grid=grid,
        in_specs=[
            pl.BlockSpec((BB, i), lambda j: (j, 0)),
            pl.BlockSpec(mu.shape, lambda j: (0, 0)),
            pl.BlockSpec(sigma.shape, lambda j: (0, 0)),
            pl.BlockSpec(rule_masks.shape, lambda j: (0, 0)),
        ],
        out_specs=pl.BlockSpec((BB, r), lambda j: (j, 0)),
        out_shape=jax.ShapeDtypeStruct((b, r), jnp.float32),
        scratch_shapes=[
            pltpu.VMEM((r, 2 * i), jnp.float32),
            pltpu.VMEM((1, r), jnp.float32),
        ],
    )(x, mu, sigma, rule_masks.astype(jnp.int32))
